# baseline (speedup 1.0000x reference)
import functools

import jax
import jax.numpy as jnp
from jax import lax
from jax.experimental import pallas as pl
from jax.experimental.pallas import tpu as pltpu
from jax.experimental.pallas import tpu_sc as plsc

E = 16
NC, NS = 2, 16
NW = NC * NS


@functools.lru_cache(maxsize=None)
def _convert_sc(V, E):
    CB = 512
    NB = V // CB
    TAIL = V - NB * CB
    mesh = plsc.VectorSubcoreMesh(core_axis_name="c", subcore_axis_name="s")

    @functools.partial(
        pl.kernel,
        out_type=jax.ShapeDtypeStruct((V * E,), jnp.float32),
        mesh=mesh,
        scratch_types=[
            pltpu.VMEM((E, CB), jnp.float32),
            pltpu.VMEM((E, TAIL), jnp.float32),
            pltpu.VMEM((CB * E,), jnp.float32),
        ],
        compiler_params=pltpu.CompilerParams(
            use_tc_tiling_on_sc=True, needs_layout_passes=False
        ),
    )
    def conv(tt_hbm, out_hbm, in_v, in_tail_v, out_v):
        wid = lax.axis_index("c") * NS + lax.axis_index("s")
        lanes = lax.iota(jnp.int32, 16)

        def do_block(col0, ncols, buf):
            pltpu.sync_copy(tt_hbm.at[:, pl.ds(col0, ncols)], buf)

            def vloop(i, _):
                for u in range(4):
                    vl = i * 4 + u
                    vec = plsc.load_gather(
                        buf, [lanes, jnp.full((16,), vl, jnp.int32)]
                    )
                    out_v[pl.ds(vl * E, E)] = vec
                return _

            lax.fori_loop(0, ncols // 4, vloop, 0)
            pltpu.sync_copy(
                out_v.at[pl.ds(0, ncols * E)],
                out_hbm.at[pl.ds(col0 * E, ncols * E)],
            )

        cnt = (NB - 1 - wid) // NW + 1

        def bloop(k, _):
            do_block((wid + k * NW) * CB, CB, in_v)
            return _

        lax.fori_loop(0, cnt, bloop, 0)
        if TAIL:
            @pl.when(wid == NW - 1)
            def _():
                do_block(NB * CB, TAIL, in_tail_v)

    return conv


@functools.lru_cache(maxsize=None)
def _pool_sc(B, L, V):
    RW = B // NW
    C = 16
    NCHUNK = RW // C
    IDX = C * L
    NG = IDX // 128
    UN = 8

    mesh = plsc.VectorSubcoreMesh(core_axis_name="c", subcore_axis_name="s")

    @functools.partial(
        pl.kernel,
        out_type=jax.ShapeDtypeStruct((B, E), jnp.float32),
        mesh=mesh,
        scratch_types=[
            pltpu.VMEM((IDX,), jnp.int32),
            pltpu.VMEM((IDX, E), jnp.float32),
            pltpu.VMEM((C, E), jnp.float32),
            pltpu.SemaphoreType.DMA,
        ],
        compiler_params=pltpu.CompilerParams(use_tc_tiling_on_sc=False),
    )
    def pool(x_hbm, table_hbm, out_hbm, idx_v, rows_v, pooled_v, sem):
        wid = lax.axis_index("c") * NS + lax.axis_index("s")

        def chunk_body(ci, carry):
            row0 = wid * RW + ci * C
            pltpu.sync_copy(x_hbm.at[pl.ds(row0 * L, IDX)], idx_v)
            copies = [
                pltpu.async_copy(
                    table_hbm.at[idx_v.at[pl.ds(j * 128, 128)]],
                    rows_v.at[pl.ds(j * 128, 128)],
                    sem,
                )
                for j in range(NG)
            ]
            for cp in copies:
                cp.wait()

            def row_body(r, c2):
                base = r * L

                def acc_body(i, accs):
                    o = base + i * UN
                    return tuple(accs[k] + rows_v[o + k] for k in range(UN))

                accs = lax.fori_loop(
                    0, L // UN, acc_body,
                    tuple(jnp.zeros((E,), jnp.float32) for _ in range(UN)),
                )
                s = accs[0]
                for k in range(1, UN):
                    s = s + accs[k]
                pooled_v[r] = s * (1.0 / L)
                return c2

            lax.fori_loop(0, C, row_body, 0)
            pltpu.sync_copy(pooled_v, out_hbm.at[pl.ds(row0, C)])
            return carry

        lax.fori_loop(0, NCHUNK, chunk_body, 0)

    return pool


def _mlp_body(p_ref, w1_ref, b1_ref, w2_ref, b2_ref, o_ref):
    p = p_ref[...]
    h = jnp.dot(p, w1_ref[...].T, preferred_element_type=jnp.float32)
    h = jnp.maximum(h + b1_ref[...], 0.0)
    o_ref[...] = jnp.dot(h, w2_ref[...].T, preferred_element_type=jnp.float32) + b2_ref[...]


def kernel(x, table, W1, b1, W2, b2):
    B, L = x.shape
    V, Ed = table.shape
    x2 = x.reshape(B * L)
    tlin = _convert_sc(V, Ed)(table.T)
    t3 = tlin.reshape(V, Ed)
    pooled = _pool_sc(B, L, V)(x2, t3)
    nc = W2.shape[0]
    out = pl.pallas_call(
        _mlp_body,
        out_shape=jax.ShapeDtypeStruct((B, nc), jnp.float32),
    )(pooled, W1, b1.reshape(1, -1), W2, b2.reshape(1, -1))
    return out


# speedup vs baseline: 1.2543x; 1.2543x over previous
import functools

import jax
import jax.numpy as jnp
from jax import lax
from jax.experimental import pallas as pl
from jax.experimental.pallas import tpu as pltpu
from jax.experimental.pallas import tpu_sc as plsc

E = 16
NC, NS = 2, 16
NW = NC * NS


def _convert_tc_body(in_ref, out_ref):
    v = in_ref[...]
    t = v.T
    rb, e = out_ref.shape[0], v.shape[0]
    t3 = t.reshape(rb, 8, e)
    out_ref[...] = jnp.concatenate([t3[:, k, :] for k in range(8)], axis=1)


@functools.lru_cache(maxsize=None)
def _convert_tc(V, E):
    CB = 4096
    G = -(-V // CB)
    RB = CB * E // 128
    return pl.pallas_call(
        _convert_tc_body,
        grid=(G,),
        in_specs=[pl.BlockSpec((E, CB), lambda b: (0, b))],
        out_specs=pl.BlockSpec((RB, 128), lambda b: (b, 0)),
        out_shape=jax.ShapeDtypeStruct((V * E // 128, 128), jnp.float32),
    )


@functools.lru_cache(maxsize=None)
def _pool_sc(B, L, V):
    RW = B // NW
    C = 16
    NCHUNK = RW // C
    IDX = C * L
    NG = IDX // 128
    UN = 8

    mesh = plsc.VectorSubcoreMesh(core_axis_name="c", subcore_axis_name="s")

    @functools.partial(
        pl.kernel,
        out_type=jax.ShapeDtypeStruct((B, E), jnp.float32),
        mesh=mesh,
        scratch_types=[
            pltpu.VMEM((IDX,), jnp.int32),
            pltpu.VMEM((IDX, E), jnp.float32),
            pltpu.VMEM((C, E), jnp.float32),
            pltpu.SemaphoreType.DMA,
        ],
        compiler_params=pltpu.CompilerParams(use_tc_tiling_on_sc=False),
    )
    def pool(x_hbm, table_hbm, out_hbm, idx_v, rows_v, pooled_v, sem):
        wid = lax.axis_index("c") * NS + lax.axis_index("s")

        def chunk_body(ci, carry):
            row0 = wid * RW + ci * C
            pltpu.sync_copy(x_hbm.at[pl.ds(row0 * L, IDX)], idx_v)
            copies = [
                pltpu.async_copy(
                    table_hbm.at[idx_v.at[pl.ds(j * 128, 128)]],
                    rows_v.at[pl.ds(j * 128, 128)],
                    sem,
                )
                for j in range(NG)
            ]
            for cp in copies:
                cp.wait()

            def row_body(r, c2):
                base = r * L

                def acc_body(i, accs):
                    o = base + i * UN
                    return tuple(accs[k] + rows_v[o + k] for k in range(UN))

                accs = lax.fori_loop(
                    0, L // UN, acc_body,
                    tuple(jnp.zeros((E,), jnp.float32) for _ in range(UN)),
                )
                s = accs[0]
                for k in range(1, UN):
                    s = s + accs[k]
                pooled_v[r] = s * (1.0 / L)
                return c2

            lax.fori_loop(0, C, row_body, 0)
            pltpu.sync_copy(pooled_v, out_hbm.at[pl.ds(row0, C)])
            return carry

        lax.fori_loop(0, NCHUNK, chunk_body, 0)

    return pool


def _mlp_body(p_ref, w1_ref, b1_ref, w2_ref, b2_ref, o_ref):
    p = p_ref[...]
    h = jnp.dot(p, w1_ref[...].T, preferred_element_type=jnp.float32)
    h = jnp.maximum(h + b1_ref[...], 0.0)
    o_ref[...] = jnp.dot(h, w2_ref[...].T, preferred_element_type=jnp.float32) + b2_ref[...]


def kernel(x, table, W1, b1, W2, b2):
    B, L = x.shape
    V, Ed = table.shape
    x2 = x.reshape(B * L)
    tlin = _convert_tc(V, Ed)(table.T)
    t3 = tlin.reshape(V, Ed)
    pooled = _pool_sc(B, L, V)(x2, t3)
    nc = W2.shape[0]
    out = pl.pallas_call(
        _mlp_body,
        out_shape=jax.ShapeDtypeStruct((B, nc), jnp.float32),
    )(pooled, W1, b1.reshape(1, -1), W2, b2.reshape(1, -1))
    return out


# speedup vs baseline: 1.4990x; 1.1951x over previous
import functools

import jax
import jax.numpy as jnp
from jax import lax
from jax.experimental import pallas as pl
from jax.experimental.pallas import tpu as pltpu
from jax.experimental.pallas import tpu_sc as plsc

E = 16
NC, NS = 2, 16
NW = NC * NS


def _convert_tc_body(in_ref, out_ref):
    v = in_ref[...]
    t = v.T
    rb, e = out_ref.shape[0], v.shape[0]
    t3 = t.reshape(rb, 8, e)
    out_ref[...] = jnp.concatenate([t3[:, k, :] for k in range(8)], axis=1)


@functools.lru_cache(maxsize=None)
def _convert_tc(V, E):
    CB = 4096
    G = -(-V // CB)
    RB = CB * E // 128
    return pl.pallas_call(
        _convert_tc_body,
        grid=(G,),
        in_specs=[pl.BlockSpec((E, CB), lambda b: (0, b))],
        out_specs=pl.BlockSpec((RB, 128), lambda b: (b, 0)),
        out_shape=jax.ShapeDtypeStruct((V * E // 128, 128), jnp.float32),
    )


@functools.lru_cache(maxsize=None)
def _pool_sc(B, L, V):
    RW = B // NW
    C = 16
    NCHUNK = RW // C
    IDX = C * L
    NG = IDX // 128
    UN = 8

    mesh = plsc.VectorSubcoreMesh(core_axis_name="c", subcore_axis_name="s")

    @functools.partial(
        pl.kernel,
        out_type=jax.ShapeDtypeStruct((B, E), jnp.float32),
        mesh=mesh,
        scratch_types=[
            pltpu.VMEM((IDX,), jnp.int32),
            pltpu.VMEM((IDX,), jnp.int32),
            pltpu.VMEM((IDX, E), jnp.float32),
            pltpu.VMEM((IDX, E), jnp.float32),
            pltpu.VMEM((C, E), jnp.float32),
            pltpu.SemaphoreType.DMA,
            pltpu.SemaphoreType.DMA,
        ],
        compiler_params=pltpu.CompilerParams(use_tc_tiling_on_sc=False),
    )
    def pool(x_hbm, table_hbm, out_hbm, idx0, idx1, rows0, rows1, pooled_v,
             sem0, sem1):
        wid = lax.axis_index("c") * NS + lax.axis_index("s")
        bufs = ((idx0, rows0, sem0), (idx1, rows1, sem1))

        def fire(ci, par):
            idx_v, rows_v, sem = bufs[par]
            row0 = wid * RW + ci * C
            pltpu.sync_copy(x_hbm.at[pl.ds(row0 * L, IDX)], idx_v)
            for j in range(NG):
                pltpu.async_copy(
                    table_hbm.at[idx_v.at[pl.ds(j * 128, 128)]],
                    rows_v.at[pl.ds(j * 128, 128)],
                    sem,
                )

        def consume(ci, par):
            idx_v, rows_v, sem = bufs[par]
            row0 = wid * RW + ci * C
            for j in range(NG):
                pltpu.make_async_copy(
                    table_hbm.at[idx_v.at[pl.ds(j * 128, 128)]],
                    rows_v.at[pl.ds(j * 128, 128)],
                    sem,
                ).wait()

            def row_body(r, c2):
                base = r * L

                def acc_body(i, accs):
                    o = base + i * UN
                    return tuple(accs[k] + rows_v[o + k] for k in range(UN))

                accs = lax.fori_loop(
                    0, L // UN, acc_body,
                    tuple(jnp.zeros((E,), jnp.float32) for _ in range(UN)),
                )
                s = accs[0]
                for k in range(1, UN):
                    s = s + accs[k]
                pooled_v[r] = s * (1.0 / L)
                return c2

            lax.fori_loop(0, C, row_body, 0)
            pltpu.sync_copy(pooled_v, out_hbm.at[pl.ds(row0, C)])

        fire(0, 0)

        def pair_body(k, carry):
            a = k * 2
            fire(a + 1, 1)
            consume(a, 0)

            @pl.when(a + 2 < NCHUNK)
            def _():
                fire(a + 2, 0)

            consume(a + 1, 1)
            return carry

        lax.fori_loop(0, NCHUNK // 2, pair_body, 0)

    return pool


def _mlp_body(p_ref, w1_ref, b1_ref, w2_ref, b2_ref, o_ref):
    p = p_ref[...]
    h = jnp.dot(p, w1_ref[...].T, preferred_element_type=jnp.float32)
    h = jnp.maximum(h + b1_ref[...], 0.0)
    o_ref[...] = jnp.dot(h, w2_ref[...].T, preferred_element_type=jnp.float32) + b2_ref[...]


def kernel(x, table, W1, b1, W2, b2):
    B, L = x.shape
    V, Ed = table.shape
    x2 = x.reshape(B * L)
    tlin = _convert_tc(V, Ed)(table.T)
    t3 = tlin.reshape(V, Ed)
    pooled = _pool_sc(B, L, V)(x2, t3)
    nc = W2.shape[0]
    out = pl.pallas_call(
        _mlp_body,
        out_shape=jax.ShapeDtypeStruct((B, nc), jnp.float32),
    )(pooled, W1, b1.reshape(1, -1), W2, b2.reshape(1, -1))
    return out


# speedup vs baseline: 1.6801x; 1.1208x over previous
import functools

import jax
import jax.numpy as jnp
from jax import lax
from jax.experimental import pallas as pl
from jax.experimental.pallas import tpu as pltpu
from jax.experimental.pallas import tpu_sc as plsc

E = 16
NC, NS = 2, 16
NW = NC * NS


def _convert_tc_body(in_ref, out_ref):
    v = in_ref[...]
    e = v.shape[0]
    rb = out_ref.shape[0]
    t3 = v.T.reshape(rb, 8, e)
    for k in range(8):
        out_ref[:, k * e:(k + 1) * e] = t3[:, k, :]


@functools.lru_cache(maxsize=None)
def _convert_tc(V, E):
    CB = 16384
    G = -(-V // CB)
    RB = CB * E // 128
    return pl.pallas_call(
        _convert_tc_body,
        grid=(G,),
        in_specs=[pl.BlockSpec((E, CB), lambda b: (0, b))],
        out_specs=pl.BlockSpec((RB, 128), lambda b: (b, 0)),
        out_shape=jax.ShapeDtypeStruct((V * E // 128, 128), jnp.float32),
    )


@functools.lru_cache(maxsize=None)
def _pool_sc(B, L, V):
    RW = B // NW
    C = 16
    NCHUNK = RW // C
    IDX = C * L
    NG = IDX // 128
    UN = 8

    mesh = plsc.VectorSubcoreMesh(core_axis_name="c", subcore_axis_name="s")

    @functools.partial(
        pl.kernel,
        out_type=jax.ShapeDtypeStruct((B, E), jnp.float32),
        mesh=mesh,
        scratch_types=[
            pltpu.VMEM((IDX,), jnp.int32),
            pltpu.VMEM((IDX,), jnp.int32),
            pltpu.VMEM((IDX, E), jnp.float32),
            pltpu.VMEM((IDX, E), jnp.float32),
            pltpu.VMEM((C, E), jnp.float32),
            pltpu.SemaphoreType.DMA,
            pltpu.SemaphoreType.DMA,
        ],
        compiler_params=pltpu.CompilerParams(use_tc_tiling_on_sc=False),
    )
    def pool(x_hbm, table_hbm, out_hbm, idx0, idx1, rows0, rows1, pooled_v,
             sem0, sem1):
        wid = lax.axis_index("c") * NS + lax.axis_index("s")
        bufs = ((idx0, rows0, sem0), (idx1, rows1, sem1))

        def fire(ci, par):
            idx_v, rows_v, sem = bufs[par]
            row0 = wid * RW + ci * C
            pltpu.sync_copy(x_hbm.at[pl.ds(row0 * L, IDX)], idx_v)
            for j in range(NG):
                pltpu.async_copy(
                    table_hbm.at[idx_v.at[pl.ds(j * 128, 128)]],
                    rows_v.at[pl.ds(j * 128, 128)],
                    sem,
                )

        def consume(ci, par):
            idx_v, rows_v, sem = bufs[par]
            row0 = wid * RW + ci * C
            for j in range(NG):
                pltpu.make_async_copy(
                    table_hbm.at[idx_v.at[pl.ds(j * 128, 128)]],
                    rows_v.at[pl.ds(j * 128, 128)],
                    sem,
                ).wait()

            def row_body(r, c2):
                base = r * L

                def acc_body(i, accs):
                    o = base + i * UN
                    return tuple(accs[k] + rows_v[o + k] for k in range(UN))

                accs = lax.fori_loop(
                    0, L // UN, acc_body,
                    tuple(jnp.zeros((E,), jnp.float32) for _ in range(UN)),
                )
                s = accs[0]
                for k in range(1, UN):
                    s = s + accs[k]
                pooled_v[r] = s * (1.0 / L)
                return c2

            lax.fori_loop(0, C, row_body, 0)
            pltpu.sync_copy(pooled_v, out_hbm.at[pl.ds(row0, C)])

        fire(0, 0)

        def pair_body(k, carry):
            a = k * 2
            fire(a + 1, 1)
            consume(a, 0)

            @pl.when(a + 2 < NCHUNK)
            def _():
                fire(a + 2, 0)

            consume(a + 1, 1)
            return carry

        lax.fori_loop(0, NCHUNK // 2, pair_body, 0)

    return pool


def _mlp_body(p_ref, w1_ref, b1_ref, w2_ref, b2_ref, o_ref):
    p = p_ref[...]
    h = jnp.dot(p, w1_ref[...].T, preferred_element_type=jnp.float32)
    h = jnp.maximum(h + b1_ref[...], 0.0)
    o_ref[...] = jnp.dot(h, w2_ref[...].T, preferred_element_type=jnp.float32) + b2_ref[...]


def kernel(x, table, W1, b1, W2, b2):
    B, L = x.shape
    V, Ed = table.shape
    x2 = x.reshape(B * L)
    tlin = _convert_tc(V, Ed)(table.T)
    t3 = tlin.reshape(V, Ed)
    pooled = _pool_sc(B, L, V)(x2, t3)
    nc = W2.shape[0]
    out = pl.pallas_call(
        _mlp_body,
        out_shape=jax.ShapeDtypeStruct((B, nc), jnp.float32),
    )(pooled, W1, b1.reshape(1, -1), W2, b2.reshape(1, -1))
    return out


# speedup vs baseline: 1.6893x; 1.0054x over previous
import functools

import jax
import jax.numpy as jnp
from jax import lax
from jax.experimental import pallas as pl
from jax.experimental.pallas import tpu as pltpu
from jax.experimental.pallas import tpu_sc as plsc

E = 16
NC, NS = 2, 16
NW = NC * NS


def _convert_tc_body(in_ref, out_ref):
    v = in_ref[...]
    e = v.shape[0]
    rb = out_ref.shape[0]
    t3 = v.T.reshape(rb, 8, e)
    for k in range(8):
        out_ref[:, k * e:(k + 1) * e] = t3[:, k, :]


@functools.lru_cache(maxsize=None)
def _convert_tc(V, E):
    CB = 32768
    G = -(-V // CB)
    RB = CB * E // 128
    return pl.pallas_call(
        _convert_tc_body,
        grid=(G,),
        in_specs=[pl.BlockSpec((E, CB), lambda b: (0, b))],
        out_specs=pl.BlockSpec((RB, 128), lambda b: (b, 0)),
        out_shape=jax.ShapeDtypeStruct((V * E // 128, 128), jnp.float32),
    )


@functools.lru_cache(maxsize=None)
def _pool_sc(B, L, V):
    RW = B // NW
    C = 16
    NCHUNK = RW // C
    IDX = C * L
    NG = IDX // 128
    UN = 8

    mesh = plsc.VectorSubcoreMesh(core_axis_name="c", subcore_axis_name="s")

    @functools.partial(
        pl.kernel,
        out_type=jax.ShapeDtypeStruct((B, E), jnp.float32),
        mesh=mesh,
        scratch_types=[
            pltpu.VMEM((IDX,), jnp.int32),
            pltpu.VMEM((IDX,), jnp.int32),
            pltpu.VMEM((IDX, E), jnp.float32),
            pltpu.VMEM((IDX, E), jnp.float32),
            pltpu.VMEM((C, E), jnp.float32),
            pltpu.SemaphoreType.DMA,
            pltpu.SemaphoreType.DMA,
        ],
        compiler_params=pltpu.CompilerParams(use_tc_tiling_on_sc=False),
    )
    def pool(x_hbm, table_hbm, out_hbm, idx0, idx1, rows0, rows1, pooled_v,
             sem0, sem1):
        wid = lax.axis_index("c") * NS + lax.axis_index("s")
        bufs = ((idx0, rows0, sem0), (idx1, rows1, sem1))

        def fire(ci, par):
            idx_v, rows_v, sem = bufs[par]
            row0 = wid * RW + ci * C
            pltpu.sync_copy(x_hbm.at[pl.ds(row0 * L, IDX)], idx_v)
            for j in range(NG):
                pltpu.async_copy(
                    table_hbm.at[idx_v.at[pl.ds(j * 128, 128)]],
                    rows_v.at[pl.ds(j * 128, 128)],
                    sem,
                )

        def consume(ci, par):
            idx_v, rows_v, sem = bufs[par]
            row0 = wid * RW + ci * C
            for j in range(NG):
                pltpu.make_async_copy(
                    table_hbm.at[idx_v.at[pl.ds(j * 128, 128)]],
                    rows_v.at[pl.ds(j * 128, 128)],
                    sem,
                ).wait()

            def row_body(r, c2):
                base = r * L

                def acc_body(i, accs):
                    o = base + i * UN
                    return tuple(accs[k] + rows_v[o + k] for k in range(UN))

                accs = lax.fori_loop(
                    0, L // UN, acc_body,
                    tuple(jnp.zeros((E,), jnp.float32) for _ in range(UN)),
                )
                s = accs[0]
                for k in range(1, UN):
                    s = s + accs[k]
                pooled_v[r] = s * (1.0 / L)
                return c2

            lax.fori_loop(0, C, row_body, 0)
            pltpu.sync_copy(pooled_v, out_hbm.at[pl.ds(row0, C)])

        fire(0, 0)

        def pair_body(k, carry):
            a = k * 2
            fire(a + 1, 1)
            consume(a, 0)

            @pl.when(a + 2 < NCHUNK)
            def _():
                fire(a + 2, 0)

            consume(a + 1, 1)
            return carry

        lax.fori_loop(0, NCHUNK // 2, pair_body, 0)

    return pool


def _mlp_body(p_ref, w1_ref, b1_ref, w2_ref, b2_ref, o_ref):
    p = p_ref[...]
    h = jnp.dot(p, w1_ref[...].T, preferred_element_type=jnp.float32)
    h = jnp.maximum(h + b1_ref[...], 0.0)
    o_ref[...] = jnp.dot(h, w2_ref[...].T, preferred_element_type=jnp.float32) + b2_ref[...]


def kernel(x, table, W1, b1, W2, b2):
    B, L = x.shape
    V, Ed = table.shape
    x2 = x.reshape(B * L)
    tlin = _convert_tc(V, Ed)(table.T)
    t3 = tlin.reshape(V, Ed)
    pooled = _pool_sc(B, L, V)(x2, t3)
    nc = W2.shape[0]
    out = pl.pallas_call(
        _mlp_body,
        out_shape=jax.ShapeDtypeStruct((B, nc), jnp.float32),
    )(pooled, W1, b1.reshape(1, -1), W2, b2.reshape(1, -1))
    return out
